# baseline (speedup 1.0000x reference)
import functools

import jax
import jax.numpy as jnp
from jax import lax
from jax.experimental import pallas as pl
from jax.experimental.pallas import tpu as pltpu
from jax.experimental.pallas import tpu_sc as plsc

_LAMBDA = 0.5
_K = 32
_TAU = 0.8
_B = 128
_N = 8192
_BLK = 32
_GRID = _B // _BLK
_IMIN = -2147483648
_NC = 2
_NS = 16
_NW = _NC * _NS
_RPW = _B // _NW
_NBINS = 2048
_CHUNKS = _N // 16


def _scalar(x):
    if x.ndim == 0:
        return x
    return lax.squeeze(lax.slice(x, (0,), (1,)), (0,))


def _sc_mse_body(g_hbm, p_hbm, out_hbm, g_v, p_v, hist, collk, collv, outv,
                 sem_g, sem_p):
    wid = lax.axis_index("s") * _NC + lax.axis_index("c")
    base_row = wid * _RPW
    ga = pltpu.async_copy(g_hbm.at[pl.ds(base_row, _RPW)], g_v, sem_g)
    pa = pltpu.async_copy(p_hbm.at[pl.ds(base_row, _RPW)], p_v, sem_p)
    ga.wait()

    lanes = lax.iota(jnp.int32, 16)
    ones = jnp.ones((16,), jnp.int32)
    msew = jnp.float32(0.0)

    for r in range(_RPW):
        @plsc.parallel_loop(0, _NBINS // 16, unroll=8)
        def _zero(i):
            hist[pl.ds(i * 16, 16)] = jnp.zeros((16,), jnp.int32)

        @plsc.parallel_loop(0, _CHUNKS, unroll=8)
        def _hist(i):
            gv = g_v[r, pl.ds(i * 16, 16)]
            b = plsc.bitcast(gv, jnp.int32)
            ukey = b ^ (jnp.right_shift(b, 31) | jnp.int32(_IMIN))
            binidx = lax.shift_right_logical(ukey, 21)
            plsc.addupdate_scatter(hist, [binidx], ones)

        def _scond(c):
            return jnp.logical_not(c[4])

        def _sbody(c):
            i, tot, cutbin, cntab, _ = c
            h = hist[pl.ds(i * 16, 16)]
            cs = plsc.cumsum(h)
            ctot = _scalar(lax.reduce_max(cs, (0,)))
            a = (tot + ctot - cs) + h
            ge = a >= jnp.int32(_K)
            npos = _scalar(plsc.all_reduce_population_count(ge))
            crossed = npos > 0
            lstar = npos - 1
            al = _scalar(lax.reduce_max(
                jnp.where(lanes == lstar, a, 0), (0,)))
            hl = _scalar(lax.reduce_max(
                jnp.where(lanes == lstar, h, 0), (0,)))
            return (i - 1, tot + ctot,
                    jnp.where(crossed, i * 16 + lstar, cutbin),
                    jnp.where(crossed, al - hl, cntab),
                    crossed)

        _, _, cutbin, cntab, _ = lax.while_loop(
            _scond, _sbody,
            (jnp.int32(_NBINS // 16 - 1), jnp.int32(0), jnp.int32(0),
             jnp.int32(0), jnp.bool_(False)))

        u_lo = jnp.broadcast_to(lax.shift_left(cutbin, 21), (16,))
        u_hi = jnp.broadcast_to(lax.shift_left(cutbin + 1, 21), (16,))
        lo_f = plsc.bitcast(
            jnp.where(u_lo < 0, u_lo ^ jnp.int32(_IMIN), ~u_lo), jnp.float32)
        hi_f = plsc.bitcast(
            jnp.where(u_hi < 0, u_hi ^ jnp.int32(_IMIN), ~u_hi), jnp.float32)

        if r == 0:
            pa.wait()

        ninf = jnp.full((16,), jnp.float32(-jnp.inf))

        @plsc.parallel_loop(0, _CHUNKS, unroll=8,
                            carry=(jnp.zeros((16,), jnp.int32),
                                   jnp.zeros((16,), jnp.float32)))
        def _collect(i, c):
            offv, acc = c
            gv = g_v[r, pl.ds(i * 16, 16)]
            pv = p_v[r, pl.ds(i * 16, 16)]
            ge_lo = gv >= lo_f
            ge_hi = gv >= hi_f
            eq = jnp.logical_and(ge_lo, jnp.logical_not(ge_hi))
            d = pv - gv
            d2 = d * d
            acc = acc + jnp.where(ge_hi, d2, jnp.float32(0.0))
            gk = jnp.where(eq, gv, ninf)
            cnt = plsc.all_reduce_population_count(eq)
            pos = jnp.where(cnt > 0, offv + lanes, lanes + jnp.int32(_N))
            plsc.store_scatter(collk, [pos], gk)
            plsc.store_scatter(collv, [pos], d2)
            offv = offv + lax.shift_left(jnp.minimum(cnt, 1), 4)
            return offv, acc

        offv, acc = _collect
        off = _scalar(offv)

        s0 = plsc.sort_key_val(collk[pl.ds(0, 16)], collv[pl.ds(0, 16)],
                               descending=True)
        ak, av = s0[0], s0[1]
        bk = ninf
        bv = jnp.zeros((16,), jnp.float32)

        def _mcond(c):
            return c[0] * 16 < off

        def _mbody(c):
            j, ak, av, bk, bv = c
            s = plsc.sort_key_val(collk[pl.ds(j * 16, 16)],
                                  collv[pl.ds(j * 16, 16)],
                                  descending=False)
            ck, cv = s[0], s[1]
            wa = ak >= ck
            hk = jnp.where(wa, ak, ck)
            hv = jnp.where(wa, av, cv)
            lk = jnp.where(wa, ck, ak)
            lv = jnp.where(wa, cv, av)
            s1 = plsc.sort_key_val(hk, hv, descending=True)
            s2 = plsc.sort_key_val(lk, lv, descending=False)
            wb = bk >= s2[0]
            h2k = jnp.where(wb, bk, s2[0])
            h2v = jnp.where(wb, bv, s2[1])
            s3 = plsc.sort_key_val(h2k, h2v, descending=True)
            return j + 1, s1[0], s1[1], s3[0], s3[1]

        _, ak, av, bk, bv = lax.while_loop(
            _mcond, _mbody, (jnp.int32(1), ak, av, bk, bv))

        r_need = jnp.int32(_K) - cntab
        msecut = (_scalar(lax.reduce_sum(
                      jnp.where(lanes < r_need, av, 0.0), (0,)))
                  + _scalar(lax.reduce_sum(
                      jnp.where(lanes + 16 < r_need, bv, 0.0), (0,))))
        msew = msew + _scalar(lax.reduce_sum(acc, (0,))) + msecut

    outv[...] = jnp.where(lanes == 0, msew, jnp.float32(0.0))
    pltpu.sync_copy(outv, out_hbm.at[wid])


def _sc_mse_partials(gamma, pred):
    mesh = plsc.VectorSubcoreMesh(core_axis_name="c", subcore_axis_name="s")
    run = pl.kernel(
        _sc_mse_body,
        out_type=jax.ShapeDtypeStruct((_NW, 16), jnp.float32),
        mesh=mesh,
        scratch_types=[
            pltpu.VMEM((_RPW, _N), jnp.float32),
            pltpu.VMEM((_RPW, _N), jnp.float32),
            pltpu.VMEM((_NBINS,), jnp.int32),
            pltpu.VMEM((_N + 32,), jnp.float32),
            pltpu.VMEM((_N + 32,), jnp.float32),
            pltpu.VMEM((16,), jnp.float32),
            pltpu.SemaphoreType.DMA,
            pltpu.SemaphoreType.DMA,
        ],
        compiler_params=pltpu.CompilerParams(needs_layout_passes=False),
    )
    return run(gamma, pred)


def _stats_body(p_ref, r_ref, g_ref, out_ref, acc_ref):
    i = pl.program_id(0)

    @pl.when(i == 0)
    def _init():
        acc_ref[0] = 0.0
        acc_ref[1] = 0.0

    g = g_ref[...]
    p = p_ref[...]
    inv_tau = jnp.float32(1.0 / _TAU)

    gmax = jnp.max(g, axis=1, keepdims=True)
    pmax = jnp.max(p, axis=1, keepdims=True)
    eg = jnp.exp((g - gmax) * inv_tau)
    ep = jnp.exp((p - pmax) * inv_tau)
    zg = jnp.sum(eg, axis=1, keepdims=True)
    zp = jnp.sum(ep, axis=1, keepdims=True)
    s_raw = jnp.sum(eg * (g - p), axis=1, keepdims=True)

    link = jnp.sum((r_ref[...] - gmax) ** 2)
    kl = jnp.sum(s_raw / (zg * _TAU) + (pmax - gmax) * inv_tau
                 + jnp.log(zp / zg))

    acc_ref[0] += link
    acc_ref[1] += kl

    @pl.when(i == _GRID - 1)
    def _fin():
        lane4 = lax.broadcasted_iota(jnp.int32, (1, 4), 1)
        out_ref[...] = jnp.where(
            lane4 == 0, acc_ref[0],
            jnp.where(lane4 == 1, acc_ref[1], 0.0))


def _combine_body(sc_ref, st_ref, out_ref):
    mse = jnp.sum(sc_ref[...][:, 0:1])
    st = st_ref[...]
    link = st[0, 0]
    kl = st[0, 1]
    total = (mse / jnp.float32(_B * _K)
             + _LAMBDA * link / _B
             + (_TAU * _TAU / _B) * kl)
    out_ref[...] = total.reshape((1, 1))


@jax.jit
def kernel(pred_logits, rsrp_pred, gamma_true):
    sc_out = _sc_mse_partials(gamma_true, pred_logits)
    stats = pl.pallas_call(
        _stats_body,
        grid=(_GRID,),
        in_specs=[
            pl.BlockSpec((_BLK, _N), lambda i: (i, 0)),
            pl.BlockSpec((_BLK, 1), lambda i: (i, 0)),
            pl.BlockSpec((_BLK, _N), lambda i: (i, 0)),
        ],
        out_specs=pl.BlockSpec((1, 4), lambda i: (0, 0)),
        out_shape=jax.ShapeDtypeStruct((1, 4), jnp.float32),
        scratch_shapes=[pltpu.SMEM((2,), jnp.float32)],
    )(pred_logits, rsrp_pred, gamma_true)
    out = pl.pallas_call(
        _combine_body,
        out_shape=jax.ShapeDtypeStruct((1, 1), jnp.float32),
    )(sc_out, stats)
    return out[0, 0]
